# speedup vs baseline: 2.2124x; 2.2124x over previous
import functools

import jax
import jax.numpy as jnp
from jax import lax
from jax.experimental import pallas as pl
from jax.experimental.pallas import tpu as pltpu
from jax.experimental.pallas import tpu_sc as plsc

UP_TO = 512
DIM = 64
D = DIM * DIM

_NC = 2
_NS = 16
_NW = _NC * _NS

_K = 8
_NBUF = 2


def _build_table_kernel(prim_ref, table_ref):
    b0 = prim_ref[0]
    b1 = prim_ref[1]
    table_ref[pl.ds(0, 2)] = prim_ref[pl.ds(0, 2)]
    t2 = jnp.dot(b1, b0, preferred_element_type=jnp.float32)
    table_ref[pl.ds(2, 1)] = t2[None]
    filled = 3
    while filled < UP_TO:
        p0 = (filled + 1) // 2
        s = filled - p0
        parents = table_ref[pl.ds(p0, s)]
        pf = parents.reshape(s * DIM, DIM)
        c1 = jnp.dot(pf, b1, preferred_element_type=jnp.float32)
        c0 = jnp.dot(pf, b0, preferred_element_type=jnp.float32)
        ch = jnp.stack(
            [c1.reshape(s, DIM, DIM), c0.reshape(s, DIM, DIM)], axis=1
        ).reshape(2 * s, DIM, DIM)
        start = 2 * p0 - 1
        cnt = min(2 * s, UP_TO - start)
        table_ref[pl.ds(start, cnt)] = ch[:cnt] if cnt != 2 * s else ch
        filled = start + cnt


def _build_table(primitives):
    return pl.pallas_call(
        _build_table_kernel,
        out_shape=jax.ShapeDtypeStruct((UP_TO, DIM, DIM), jnp.float32),
    )(primitives)


def _make_gather(n_lookups):
    b_per_w = n_lookups // _NW
    n_chunk = b_per_w // _K
    mesh = plsc.VectorSubcoreMesh(core_axis_name="c", subcore_axis_name="s")

    @functools.partial(
        pl.kernel,
        out_type=jax.ShapeDtypeStruct((n_lookups, D), jnp.float32),
        mesh=mesh,
        scratch_types=[
            pltpu.VMEM((n_chunk, _K), jnp.int32),
            *[pltpu.VMEM((_K, D), jnp.float32) for _ in range(_NBUF)],
            *[pltpu.SemaphoreType.DMA for _ in range(2 * _NBUF)],
        ],
    )
    def gather_kernel(table_hbm, idx_hbm, out_hbm, idx_v, *bufs_sems):
        bufs = bufs_sems[:_NBUF]
        sins = bufs_sems[_NBUF:2 * _NBUF]
        souts = bufs_sems[2 * _NBUF:]
        wid = lax.axis_index("s") * _NC + lax.axis_index("c")
        base = wid * b_per_w
        pltpu.sync_copy(idx_hbm.at[wid], idx_v)

        for b in range(_NBUF):
            pltpu.async_copy(table_hbm.at[idx_v.at[b]], bufs[b], sins[b])

        @pl.loop(0, n_chunk, step=_NBUF)
        def _(c):
            for b in range(_NBUF):
                cur = c + b
                pltpu.make_async_copy(
                    table_hbm.at[idx_v.at[cur]], bufs[b], sins[b]
                ).wait()
                out_slice = out_hbm.at[pl.ds(base + cur * _K, _K)]
                pltpu.async_copy(bufs[b], out_slice, souts[b])
                pltpu.make_async_copy(bufs[b], out_slice, souts[b]).wait()
                nxt = cur + _NBUF

                @pl.when(nxt < n_chunk)
                def _():
                    pltpu.async_copy(
                        table_hbm.at[idx_v.at[nxt]], bufs[b], sins[b]
                    )

    return gather_kernel


def kernel(primitives, node_positions):
    n = node_positions.shape[0]
    table = _build_table(primitives)
    table2d = table.reshape(UP_TO, D)
    idx = (node_positions - 1).astype(jnp.int32).reshape(_NW, n // (_NW * _K), _K)
    out = _make_gather(n)(table2d, idx)
    return out.reshape(n, DIM, DIM)
